# Initial kernel scaffold; baseline (speedup 1.0000x reference)
import jax
import jax.numpy as jnp
from jax.experimental import pallas as pl


def kernel(x, edge_index, edge_attr, ne_w1, ne_b1, ne_w2, ne_b2, ee_w1, ee_b1, ee_w2, ee_b2, eu_w1, eu_b1, eu_w2, eu_b2, nu_w1, nu_b1, nu_w2, nu_b2, de_w1, de_b1, de_w2, de_b2, de_w3, de_b3):
    raise NotImplementedError("write your pallas kernel here")


# speedup vs baseline: 1.8629x; 1.8629x over previous
import functools

import jax
import jax.numpy as jnp
from jax import lax
from jax.experimental import pallas as pl
from jax.experimental.pallas import tpu as pltpu
from jax.experimental.pallas import tpu_sc as plsc

N_NODES_C = 100000
N_EDGES_C = 1600000
HID = 64

NC, NS = 2, 16
NW = NC * NS


_NB = 2000
_EB = 2000


def _dot(a, b):
    return jnp.dot(a, b, preferred_element_type=jnp.float32)


def _enc_node_body(x_ref, w1r_ref, b1_ref, w2t_ref, b2_ref,
                   nbt_ref, nb1_ref, at_ref, bt_ref,
                   enc_ref, f_ref, pq_ref):
    x = x_ref[...]
    h = jax.nn.relu(x * w1r_ref[...] + b1_ref[...])
    enc = jax.nn.relu(_dot(h, w2t_ref[...]) + b2_ref[...])
    enc_ref[...] = enc
    f_ref[...] = _dot(enc, nbt_ref[...]) + nb1_ref[...]
    pq_ref[...] = jnp.concatenate(
        [_dot(enc, at_ref[...]), _dot(enc, bt_ref[...])], axis=1)


def _enc_edge_body(ea_ref, w1t_ref, b1_ref, w2t_ref, b2_ref, dt_ref, eb1_ref,
                   enc_ref, g_ref):
    h = jax.nn.relu(_dot(ea_ref[...], w1t_ref[...]) + b1_ref[...])
    enc = jax.nn.relu(_dot(h, w2t_ref[...]) + b2_ref[...])
    enc_ref[...] = enc
    g_ref[...] = _dot(enc, dt_ref[...]) + eb1_ref[...]


def _edge_mlp_body(s1_ref, s2_ref, eh_ref, g_ref, ct_ref, w2t_ref, b2_ref,
                   out_ref):
    t = s1_ref[:, :HID] + s2_ref[:, HID:] + g_ref[...] \
        + _dot(eh_ref[...], ct_ref[...])
    h = jax.nn.relu(t)
    out_ref[...] = jax.nn.relu(_dot(h, w2t_ref[...]) + b2_ref[...])


def _node_mlp_body(xh_ref, recv_ref, f_ref, nat_ref, nct_ref, w2t_ref, b2_ref,
                   at_ref, bt_ref, xh_o, pq_o):
    t = f_ref[...] + _dot(xh_ref[...], nat_ref[...]) \
        + _dot(recv_ref[...], nct_ref[...])
    h = jax.nn.relu(t)
    xh = jax.nn.relu(_dot(h, w2t_ref[...]) + b2_ref[...])
    xh_o[...] = xh
    pq_o[...] = jnp.concatenate(
        [_dot(xh, at_ref[...]), _dot(xh, bt_ref[...])], axis=1)


def _decode_body(xh_ref, w1t_ref, b1_ref, w2t_ref, b2_ref, w3t_ref, b3_ref,
                 out_ref):
    h = jax.nn.relu(_dot(xh_ref[...], w1t_ref[...]) + b1_ref[...])
    h = jax.nn.relu(_dot(h, w2t_ref[...]) + b2_ref[...])
    out_ref[...] = _dot(h, w3t_ref[...]) + b3_ref[...]


def _row_spec(nrows, ncols):
    return pl.BlockSpec((nrows, ncols), lambda i: (i, 0))


def _full_spec(shape):
    return pl.BlockSpec(shape, lambda i: tuple(0 for _ in shape))


def _tc_call(body, grid, in_arrays, in_blocked, out_shapes, block_rows):
    in_specs = []
    for a, blocked in zip(in_arrays, in_blocked):
        if blocked:
            in_specs.append(_row_spec(blocked, a.shape[1]))
        else:
            in_specs.append(_full_spec(a.shape))
    out_specs = [_row_spec(block_rows, s[1]) for (s, _) in out_shapes]
    return pl.pallas_call(
        body,
        grid=(grid,),
        in_specs=in_specs,
        out_specs=out_specs if len(out_specs) > 1 else out_specs[0],
        out_shape=[jax.ShapeDtypeStruct(s, d) for (s, d) in out_shapes]
        if len(out_shapes) > 1
        else jax.ShapeDtypeStruct(out_shapes[0][0], out_shapes[0][1]),
        compiler_params=pltpu.CompilerParams(
            dimension_semantics=("arbitrary",)),
    )(*in_arrays)


_CH = 80
_NSUP = 25
_SUP = _CH * _NSUP

_E_PER_W = N_EDGES_C // NW
_GCH_N = _E_PER_W // _CH


def _sc_gather_kernel(t_hbm, row_hbm, col_hbm, s1_hbm, s2_hbm,
                      idxr_v, idxc_v, bufp0, bufp1, bufq0, bufq1,
                      gsem0, gsem1, wsem0, wsem1):
    wid = lax.axis_index("s") * NC + lax.axis_index("c")
    base = wid * _E_PER_W
    bufp = (bufp0, bufp1)
    bufq = (bufq0, bufq1)
    gsem = (gsem0, gsem1)
    wsem = (wsem0, wsem1)

    def load_super(s):
        g = wid * (_GCH_N // _NSUP) + s
        pltpu.sync_copy(row_hbm.at[g], idxr_v.at[s % 2])
        pltpu.sync_copy(col_hbm.at[g], idxc_v.at[s % 2])

    def issue_gather(j, slot):
        sup = j // _NSUP
        jj = j % _NSUP
        ir = idxr_v.at[sup % 2].at[jj]
        ic = idxc_v.at[sup % 2].at[jj]
        pltpu.async_copy(t_hbm.at[ir], bufp[slot], gsem[slot])
        pltpu.async_copy(t_hbm.at[ic], bufq[slot], gsem[slot])

    def wait_gather(j, slot):
        sup = j // _NSUP
        jj = j % _NSUP
        pltpu.make_async_copy(
            t_hbm.at[idxr_v.at[sup % 2].at[jj]], bufp[slot], gsem[slot]).wait()
        pltpu.make_async_copy(
            t_hbm.at[idxc_v.at[sup % 2].at[jj]], bufq[slot], gsem[slot]).wait()

    def do_write(j, slot):
        off = base + j * _CH
        pltpu.async_copy(bufp[slot], s1_hbm.at[pl.ds(off, _CH)], wsem[slot])
        pltpu.async_copy(bufq[slot], s2_hbm.at[pl.ds(off, _CH)], wsem[slot])
        pltpu.make_async_copy(
            bufp[slot], s1_hbm.at[pl.ds(off, _CH)], wsem[slot]).wait()
        pltpu.make_async_copy(
            bufq[slot], s2_hbm.at[pl.ds(off, _CH)], wsem[slot]).wait()

    load_super(0)
    issue_gather(0, 0)

    def body2(jh, _):
        for half in range(2):
            j = jh * 2 + half
            slot = half
            nslot = 1 - half
            nxt = j + 1

            @pl.when((nxt % _NSUP == 0) & (nxt < _GCH_N))
            def _():
                load_super(nxt // _NSUP)

            @pl.when(nxt < _GCH_N)
            def _():
                issue_gather(nxt, nslot)
            wait_gather(j, slot)
            do_write(j, slot)
        return _

    lax.fori_loop(0, _GCH_N // 2, body2, None)
    j = _GCH_N - 1
    wait_gather(j, 0)
    do_write(j, 0)


@functools.cache
def _sc_kernels():
    mesh = plsc.VectorSubcoreMesh(core_axis_name="c", subcore_axis_name="s",
                                  num_cores=NC, num_subcores=NS)
    gather = pl.kernel(
        _sc_gather_kernel,
        out_type=[jax.ShapeDtypeStruct((N_EDGES_C, 2 * HID), jnp.float32),
                  jax.ShapeDtypeStruct((N_EDGES_C, 2 * HID), jnp.float32)],
        mesh=mesh,
        scratch_types=[
            pltpu.VMEM((2, _NSUP, _CH), jnp.int32),
            pltpu.VMEM((2, _NSUP, _CH), jnp.int32),
            pltpu.VMEM((_CH, 2 * HID), jnp.float32),
            pltpu.VMEM((_CH, 2 * HID), jnp.float32),
            pltpu.VMEM((_CH, 2 * HID), jnp.float32),
            pltpu.VMEM((_CH, 2 * HID), jnp.float32),
            pltpu.SemaphoreType.DMA,
            pltpu.SemaphoreType.DMA,
            pltpu.SemaphoreType.DMA,
            pltpu.SemaphoreType.DMA,
        ],
    )
    return gather


def _gather_pq(pq, row, col):
    row3 = row.reshape(-1, _NSUP, _CH)
    col3 = col.reshape(-1, _NSUP, _CH)
    return _sc_kernels()(pq, row3, col3)


def kernel(x, edge_index, edge_attr,
           ne_w1, ne_b1, ne_w2, ne_b2,
           ee_w1, ee_b1, ee_w2, ee_b2,
           eu_w1, eu_b1, eu_w2, eu_b2,
           nu_w1, nu_b1, nu_w2, nu_b2,
           de_w1, de_b1, de_w2, de_b2, de_w3, de_b3):
    n = x.shape[0]
    e = edge_attr.shape[0]
    row = edge_index[0]
    col = edge_index[1]

    at_ = eu_w1[:, 0:HID].T
    bt_ = eu_w1[:, HID:2 * HID].T
    ct_ = eu_w1[:, 2 * HID:3 * HID].T
    dt_ = eu_w1[:, 3 * HID:4 * HID].T
    nat_ = nu_w1[:, 0:HID].T
    nbt_ = nu_w1[:, HID:2 * HID].T
    nct_ = nu_w1[:, 2 * HID:3 * HID].T

    b = lambda v: v.reshape(1, -1)

    f32 = jnp.float32
    enc_x, f_, pq = _tc_call(
        _enc_node_body, n // _NB,
        [x, ne_w1.T, b(ne_b1), ne_w2.T, b(ne_b2), nbt_, b(nu_b1), at_, bt_],
        [_NB, 0, 0, 0, 0, 0, 0, 0, 0],
        [((n, HID), f32), ((n, HID), f32), ((n, 2 * HID), f32)], _NB)

    enc_e, g = _tc_call(
        _enc_edge_body, e // _EB,
        [edge_attr, ee_w1.T, b(ee_b1), ee_w2.T, b(ee_b2), dt_, b(eu_b1)],
        [_EB, 0, 0, 0, 0, 0, 0],
        [((e, HID), f32)] * 2, _EB)

    xh = enc_x
    eh = enc_e
    for _ in range(7):
        s1, s2 = _gather_pq(pq, row, col)
        eh = _tc_call(
            _edge_mlp_body, e // _EB,
            [s1, s2, eh, g, ct_, eu_w2.T, b(eu_b2)],
            [_EB, _EB, _EB, _EB, 0, 0, 0],
            [((e, HID), f32)], _EB)
        recv = jnp.zeros((n, HID), jnp.float32).at[col].add(eh)
        xh, pq = _tc_call(
            _node_mlp_body, n // _NB,
            [xh, recv, f_, nat_, nct_, nu_w2.T, b(nu_b2), at_, bt_],
            [_NB, _NB, _NB, 0, 0, 0, 0, 0, 0],
            [((n, HID), f32), ((n, 2 * HID), f32)], _NB)

    out = _tc_call(
        _decode_body, n // _NB,
        [xh, de_w1.T, b(de_b1), de_w2.T, b(de_b2), de_w3.T, b(de_b3)],
        [_NB, 0, 0, 0, 0, 0, 0],
        [((n, 1), f32)], _NB)
    return out
